# baseline (speedup 1.0000x reference)
import functools

import jax
import jax.numpy as jnp
from jax import lax
from jax.experimental import pallas as pl
from jax.experimental.pallas import tpu as pltpu
from jax.experimental.pallas import tpu_sc as plsc

NC = 2
NS = 16
NW = NC * NS

CHUNK = 128


def _build(B, V, D):
    b_per_w = B // NW
    n_chunks = b_per_w // CHUNK
    mesh = plsc.VectorSubcoreMesh(core_axis_name="c", subcore_axis_name="s")

    @functools.partial(
        pl.kernel,
        mesh=mesh,
        out_type=jax.ShapeDtypeStruct((B, D), jnp.float32),
        scratch_types=[
            pltpu.VMEM((CHUNK,), jnp.int32),
            pltpu.VMEM((CHUNK, D), jnp.float32),
            pltpu.SemaphoreType.DMA,
        ],
        compiler_params=pltpu.CompilerParams(use_tc_tiling_on_sc=False),
    )
    def k(idx_hbm, table_hbm, out_hbm, idx_v, rows_v, gsem):
        wid = lax.axis_index("s") * NC + lax.axis_index("c")
        base = wid * b_per_w

        def body(g, carry):
            start = base + g * CHUNK
            pltpu.sync_copy(idx_hbm.at[pl.ds(start, CHUNK)], idx_v)
            pltpu.async_copy(table_hbm.at[idx_v], rows_v, gsem).wait()
            pltpu.sync_copy(rows_v, out_hbm.at[pl.ds(start, CHUNK)])
            return carry

        lax.fori_loop(0, n_chunks, body, 0)

    return k


def kernel(token_ids, weights):
    B0, S = token_ids.shape
    V, D = weights.shape
    B = B0 * S
    idx = token_ids.reshape(B).astype(jnp.int32)
    out = _build(B, V, D)(idx, weights)
    return out.reshape(B0, S, D)


# speedup vs baseline: 1.1394x; 1.1394x over previous
import functools

import jax
import jax.numpy as jnp
from jax import lax
from jax.experimental import pallas as pl
from jax.experimental.pallas import tpu as pltpu
from jax.experimental.pallas import tpu_sc as plsc

NC = 2
NS = 16
NW = NC * NS

IDXW = 128
K = 10
STEP = K * IDXW


def _build(B, V, D):
    rows_per_w = B // NW
    idxrows_per_w = rows_per_w // IDXW
    n_iter = idxrows_per_w // K
    n_pairs = n_iter // 2
    assert rows_per_w * NW == B
    assert idxrows_per_w * IDXW == rows_per_w
    assert n_pairs * 2 == n_iter

    mesh = plsc.VectorSubcoreMesh(core_axis_name="c", subcore_axis_name="s")

    @functools.partial(
        pl.kernel,
        mesh=mesh,
        out_type=jax.ShapeDtypeStruct((B, D), jnp.float32),
        scratch_types=[
            pltpu.VMEM((idxrows_per_w, IDXW), jnp.int32),
            pltpu.VMEM((2, STEP, D), jnp.float32),
            pltpu.SemaphoreType.DMA,
            pltpu.SemaphoreType.DMA,
            pltpu.SemaphoreType.DMA,
            pltpu.SemaphoreType.DMA,
        ],
        compiler_params=pltpu.CompilerParams(use_tc_tiling_on_sc=False),
    )
    def k(idx_hbm, table_hbm, out_hbm, idx_all, rows, gsem0, gsem1,
          ssem0, ssem1):
        wid = lax.axis_index("s") * NC + lax.axis_index("c")
        idxrow0 = wid * idxrows_per_w
        base = wid * rows_per_w

        pltpu.sync_copy(idx_hbm.at[pl.ds(idxrow0, idxrows_per_w)], idx_all)

        def fire_gathers(it, buf, gsem):
            for j in range(K):
                pltpu.async_copy(
                    table_hbm.at[idx_all.at[it * K + j]],
                    rows.at[buf, pl.ds(j * IDXW, IDXW)],
                    gsem)

        def wait_gathers(buf, gsem):
            for j in range(K):
                pltpu.make_async_copy(
                    table_hbm.at[idx_all.at[0]],
                    rows.at[buf, pl.ds(j * IDXW, IDXW)],
                    gsem).wait()

        def fire_store(it, buf, ssem):
            pltpu.async_copy(
                rows.at[buf], out_hbm.at[pl.ds(base + it * STEP, STEP)],
                ssem)

        def wait_store(buf, ssem):
            pltpu.make_async_copy(
                rows.at[buf], out_hbm.at[pl.ds(base, STEP)], ssem).wait()

        fire_gathers(0, 0, gsem0)

        def body(p, carry):
            it0 = 2 * p
            it1 = it0 + 1
            @pl.when(p > 0)
            def _():
                wait_store(1, ssem1)
            fire_gathers(it1, 1, gsem1)
            wait_gathers(0, gsem0)
            fire_store(it0, 0, ssem0)
            wait_store(0, ssem0)
            @pl.when(p < n_pairs - 1)
            def _():
                fire_gathers(it0 + 2, 0, gsem0)
            wait_gathers(1, gsem1)
            fire_store(it1, 1, ssem1)
            return carry

        lax.fori_loop(0, n_pairs, body, 0)
        wait_store(1, ssem1)

    return k


def kernel(token_ids, weights):
    B0, S = token_ids.shape
    V, D = weights.shape
    B = B0 * S
    idx = token_ids.reshape(B // IDXW, IDXW).astype(jnp.int32)
    out = _build(B, V, D)(idx, weights)
    return out.reshape(B0, S, D)


# speedup vs baseline: 1.8473x; 1.6213x over previous
import functools

import jax
import jax.numpy as jnp
from jax import lax
from jax.experimental import pallas as pl
from jax.experimental.pallas import tpu as pltpu
from jax.experimental.pallas import tpu_sc as plsc

NC = 2
NS = 16
NW = NC * NS

NB = 16


def _build(B0, S, V, D):
    rows_per_w = B0 // NW
    n_iter = rows_per_w // NB
    n_pairs = n_iter // 2
    assert rows_per_w * NW == B0
    assert n_pairs * 2 * NB == rows_per_w

    mesh = plsc.VectorSubcoreMesh(core_axis_name="c", subcore_axis_name="s")

    @functools.partial(
        pl.kernel,
        mesh=mesh,
        out_type=jax.ShapeDtypeStruct((B0, S, D), jnp.float32),
        scratch_types=[
            pltpu.VMEM((rows_per_w, S), jnp.int32),
            pltpu.VMEM((2, NB, S, D), jnp.float32),
            pltpu.SemaphoreType.DMA,
            pltpu.SemaphoreType.DMA,
            pltpu.SemaphoreType.DMA,
            pltpu.SemaphoreType.DMA,
        ],
        compiler_params=pltpu.CompilerParams(use_tc_tiling_on_sc=False),
    )
    def k(idx_hbm, table_hbm, out_hbm, idx_all, rows, gsem0, gsem1,
          ssem0, ssem1):
        wid = lax.axis_index("s") * NC + lax.axis_index("c")
        base = wid * rows_per_w

        pltpu.sync_copy(idx_hbm.at[pl.ds(base, rows_per_w)], idx_all)

        def fire_gathers(it, buf, gsem):
            for j in range(NB):
                pltpu.async_copy(
                    table_hbm.at[idx_all.at[it * NB + j]],
                    rows.at[buf, j],
                    gsem)

        def wait_gathers(buf, gsem):
            for j in range(NB):
                pltpu.make_async_copy(
                    table_hbm.at[idx_all.at[0]],
                    rows.at[buf, j],
                    gsem).wait()

        def fire_store(it, buf, ssem):
            pltpu.async_copy(
                rows.at[buf], out_hbm.at[pl.ds(base + it * NB, NB)],
                ssem)

        def wait_store(buf, ssem):
            pltpu.make_async_copy(
                rows.at[buf], out_hbm.at[pl.ds(base, NB)], ssem).wait()

        fire_gathers(0, 0, gsem0)

        def body(p, carry):
            it0 = 2 * p
            it1 = it0 + 1
            @pl.when(p > 0)
            def _():
                wait_store(1, ssem1)
            fire_gathers(it1, 1, gsem1)
            wait_gathers(0, gsem0)
            fire_store(it0, 0, ssem0)
            wait_store(0, ssem0)
            @pl.when(p < n_pairs - 1)
            def _():
                fire_gathers(it0 + 2, 0, gsem0)
            wait_gathers(1, gsem1)
            fire_store(it1, 1, ssem1)
            return carry

        lax.fori_loop(0, n_pairs, body, 0)
        wait_store(1, ssem1)

    return k


def kernel(token_ids, weights):
    B0, S = token_ids.shape
    V, D = weights.shape
    return _build(B0, S, V, D)(token_ids, weights)
